# baseline (speedup 1.0000x reference)
import functools

import jax
import jax.numpy as jnp
from jax import lax
from jax.experimental import pallas as pl
from jax.experimental.pallas import tpu as pltpu
from jax.experimental.pallas import tpu_sc as plsc

V = 1000
D = 128
B = 16384

NC = 2
NS = 16
NW = NC * NS
BPW = B // NW
CHUNK = 128
NCHUNK = BPW // CHUNK

_mesh = plsc.VectorSubcoreMesh(core_axis_name="c", subcore_axis_name="s")


@functools.partial(
    pl.kernel,
    out_type=jax.ShapeDtypeStruct((B, D), jnp.float32),
    mesh=_mesh,
    scratch_types=[
        pltpu.VMEM((NCHUNK, CHUNK), jnp.int32),
        pltpu.VMEM((BPW, D), jnp.float32),
        [pltpu.SemaphoreType.DMA] * NCHUNK,
        pltpu.SemaphoreType.DMA,
    ],
)
def _gather_kernel(x_hbm, table_hbm, out_hbm, idx_v, rows_v, gsems, wsem):
    wid = lax.axis_index("s") * NC + lax.axis_index("c")
    base = wid * BPW
    pltpu.sync_copy(x_hbm.at[wid], idx_v)
    gathers = []
    for j in range(NCHUNK):
        gathers.append(
            pltpu.async_copy(
                table_hbm.at[idx_v.at[j]],
                rows_v.at[pl.ds(j * CHUNK, CHUNK)],
                gsems[j],
            )
        )
    writes = []
    for j in range(NCHUNK):
        gathers[j].wait()
        writes.append(
            pltpu.async_copy(
                rows_v.at[pl.ds(j * CHUNK, CHUNK)],
                out_hbm.at[pl.ds(base + j * CHUNK, CHUNK)],
                wsem,
            )
        )
    for c in writes:
        c.wait()


def kernel(x, table):
    x = x.astype(jnp.int32).reshape(NW, NCHUNK, CHUNK)
    return _gather_kernel(x, table)


# speedup vs baseline: 1.1727x; 1.1727x over previous
import functools

import jax
import jax.numpy as jnp
from jax import lax
from jax.experimental import pallas as pl
from jax.experimental.pallas import tpu as pltpu
from jax.experimental.pallas import tpu_sc as plsc

V = 1000
D = 128
B = 16384

NC = 2
NS = 16
NW = NC * NS
BPW = B // NW
CHUNK = 128
NCHUNK = BPW // CHUNK
LOADERS = 5
VPL = V // LOADERS

_mesh = plsc.VectorSubcoreMesh(core_axis_name="c", subcore_axis_name="s")


@functools.partial(
    pl.kernel,
    out_type=jax.ShapeDtypeStruct((B, D), jnp.float32),
    mesh=_mesh,
    scratch_types=[
        pltpu.VMEM_SHARED((V, D), jnp.float32),
        pltpu.VMEM((NCHUNK, CHUNK), jnp.int32),
        pltpu.VMEM((BPW, D), jnp.float32),
        pltpu.SemaphoreType.DMA,
    ],
)
def _gather_kernel(x_hbm, table_hbm, out_hbm, table_sp, idx_v, rows_v, sem):
    cid = lax.axis_index("c")
    sid = lax.axis_index("s")
    wid = sid * NC + cid
    base = wid * BPW

    @pl.when(sid < LOADERS)
    def _stage():
        r0 = sid * VPL
        pltpu.sync_copy(
            table_hbm.at[pl.ds(r0, VPL)], table_sp.at[pl.ds(r0, VPL)]
        )

    pltpu.sync_copy(x_hbm.at[wid], idx_v)
    plsc.subcore_barrier()

    copies = []
    for j in range(NCHUNK):
        copies.append(
            pltpu.async_copy(
                table_sp.at[idx_v.at[j]],
                rows_v.at[pl.ds(j * CHUNK, CHUNK)],
                sem,
            )
        )
    for c in copies:
        c.wait()
    pltpu.sync_copy(rows_v, out_hbm.at[pl.ds(base, BPW)])


def kernel(x, table):
    x = x.astype(jnp.int32).reshape(NW, NCHUNK, CHUNK)
    return _gather_kernel(x, table)


# speedup vs baseline: 1.2050x; 1.0276x over previous
import functools

import jax
import jax.numpy as jnp
from jax import lax
from jax.experimental import pallas as pl
from jax.experimental.pallas import tpu as pltpu
from jax.experimental.pallas import tpu_sc as plsc

V = 1000
D = 128
B = 16384

NC = 2
NS = 16
NW = NC * NS
BPW = B // NW
CHUNK = 128
NCHUNK = BPW // CHUNK
LOADERS = 5
VPL = V // LOADERS

_mesh = plsc.VectorSubcoreMesh(core_axis_name="c", subcore_axis_name="s")


@functools.partial(
    pl.kernel,
    out_type=jax.ShapeDtypeStruct((B, D), jnp.float32),
    mesh=_mesh,
    scratch_types=[
        pltpu.VMEM_SHARED((V, D), jnp.float32),
        pltpu.VMEM((NCHUNK, CHUNK), jnp.int32),
        pltpu.VMEM((BPW, D), jnp.float32),
        [pltpu.SemaphoreType.DMA] * NCHUNK,
        pltpu.SemaphoreType.DMA,
    ],
)
def _gather_kernel(x_hbm, table_hbm, out_hbm, table_sp, idx_v, rows_v, gsems, wsem):
    cid = lax.axis_index("c")
    sid = lax.axis_index("s")
    wid = sid * NC + cid
    base = wid * BPW

    @pl.when(sid < LOADERS)
    def _stage():
        r0 = sid * VPL
        pltpu.sync_copy(
            table_hbm.at[pl.ds(r0, VPL)], table_sp.at[pl.ds(r0, VPL)]
        )

    pltpu.sync_copy(x_hbm.at[wid], idx_v)
    plsc.subcore_barrier()

    gathers = []
    for j in range(NCHUNK):
        gathers.append(
            pltpu.async_copy(
                table_sp.at[idx_v.at[j]],
                rows_v.at[pl.ds(j * CHUNK, CHUNK)],
                gsems[j],
            )
        )
    writes = []
    for j in range(NCHUNK):
        gathers[j].wait()
        writes.append(
            pltpu.async_copy(
                rows_v.at[pl.ds(j * CHUNK, CHUNK)],
                out_hbm.at[pl.ds(base + j * CHUNK, CHUNK)],
                wsem,
            )
        )
    for c in writes:
        c.wait()


def kernel(x, table):
    x = x.astype(jnp.int32).reshape(NW, NCHUNK, CHUNK)
    return _gather_kernel(x, table)


# speedup vs baseline: 1.2130x; 1.0067x over previous
import functools

import jax
import jax.numpy as jnp
from jax import lax
from jax.experimental import pallas as pl
from jax.experimental.pallas import tpu as pltpu
from jax.experimental.pallas import tpu_sc as plsc

V = 1000
D = 128
B = 16384

NC = 2
NS = 16
NW = NC * NS
BPW = B // NW
CHUNK = 128
NCHUNK = BPW // CHUNK
LOADERS = 5
VPL = V // LOADERS

_mesh = plsc.VectorSubcoreMesh(core_axis_name="c", subcore_axis_name="s")


@functools.partial(
    pl.kernel,
    out_type=jax.ShapeDtypeStruct((B, D), jnp.float32),
    mesh=_mesh,
    scratch_types=[
        pltpu.VMEM_SHARED((V, D), jnp.float32),
        pltpu.VMEM((NCHUNK, CHUNK), jnp.int32),
        pltpu.VMEM((BPW, D), jnp.float32),
        pltpu.SemaphoreType.DMA,
        pltpu.SemaphoreType.DMA,
    ],
)
def _gather_kernel(x_hbm, table_hbm, out_hbm, table_sp, idx_v, rows_v, gsems, wsem):
    cid = lax.axis_index("c")
    sid = lax.axis_index("s")
    wid = sid * NC + cid
    base = wid * BPW

    @pl.when(sid < LOADERS)
    def _stage():
        r0 = sid * VPL
        pltpu.sync_copy(
            table_hbm.at[pl.ds(r0, VPL)], table_sp.at[pl.ds(r0, VPL)]
        )

    pltpu.sync_copy(x_hbm.at[wid], idx_v)
    plsc.subcore_barrier()

    def _chunk(j, carry):
        off = j * CHUNK
        pltpu.async_copy(
            table_sp.at[idx_v.at[j]],
            rows_v.at[pl.ds(off, CHUNK)],
            gsems,
        ).wait()
        pltpu.async_copy(
            rows_v.at[pl.ds(off, CHUNK)],
            out_hbm.at[pl.ds(base + off, CHUNK)],
            wsem,
        )
        return carry

    lax.fori_loop(0, NCHUNK, _chunk, 0)
    pltpu.make_async_copy(rows_v, out_hbm.at[pl.ds(base, BPW)], wsem).wait()


def kernel(x, table):
    x = x.astype(jnp.int32).reshape(NW, NCHUNK, CHUNK)
    return _gather_kernel(x, table)
